# baseline (speedup 1.0000x reference)
import functools

import jax
import jax.numpy as jnp
from jax import lax
from jax.experimental import pallas as pl
from jax.experimental.pallas import tpu as pltpu
from jax.experimental.pallas import tpu_sc as plsc

N = 10000
E = 320000
D = 128
B = 16384

NC = 2
NS = 16
NW = NC * NS
SP = 632
NP = NS * SP

ECH = 128
ENCH0 = 100
ENCH1 = 57
ENCH = max(ENCH0, ENCH1)
E0 = NS * ENCH0 * ECH
EPAD = NW * ENCH * ECH

BPW = B // NW
BCH = 128
BNCH = BPW // BCH

BN = 2000


def _sc_mesh():
  return plsc.VectorSubcoreMesh(
      core_axis_name="c", subcore_axis_name="s",
      num_cores=NC, num_subcores=NS)


@functools.partial(
    pl.kernel,
    out_type=jax.ShapeDtypeStruct((NC * 2 * NP,), jnp.float32),
    mesh=_sc_mesh(),
    scratch_types=[
        pltpu.VMEM((2 * ENCH, ECH), jnp.int32),
        pltpu.VMEM((ECH,), jnp.float32),
        pltpu.VMEM((SP,), jnp.float32),
        pltpu.VMEM_SHARED((NP,), jnp.float32),
        pltpu.VMEM_SHARED((NP,), jnp.float32),
        pltpu.SemaphoreType.DMA,
        pltpu.SemaphoreType.DMA,
    ],
)
def _deg_sc(tbl_hbm, ones_hbm, zeros_hbm, out_hbm,
            tb_v, ones_v, zbuf, dego_sh, degi_sh, s_o, s_i):
  cid = lax.axis_index("c")
  sid = lax.axis_index("s")
  wid = cid * NS + sid
  ench = jnp.where(cid == 0, ENCH0, ENCH1)
  r0 = sid * SP
  pltpu.sync_copy(zeros_hbm.at[pl.ds(0, SP)], zbuf)
  pltpu.sync_copy(zbuf, dego_sh.at[pl.ds(r0, SP)])
  pltpu.sync_copy(zbuf, degi_sh.at[pl.ds(r0, SP)])
  pltpu.sync_copy(tbl_hbm.at[wid], tb_v)
  pltpu.sync_copy(ones_hbm, ones_v)
  plsc.subcore_barrier()

  def body(j, carry):
    a = pltpu.async_copy(ones_v, dego_sh.at[tb_v.at[2 * j]], s_o, add=True)
    b = pltpu.async_copy(ones_v, degi_sh.at[tb_v.at[2 * j + 1]], s_i,
                         add=True)
    a.wait()
    b.wait()
    return carry

  lax.fori_loop(0, ench, body, 0)
  plsc.subcore_barrier()
  obase = cid * 2 * NP
  pltpu.sync_copy(dego_sh.at[pl.ds(r0, SP)], zbuf)
  pltpu.sync_copy(zbuf, out_hbm.at[pl.ds(obase + r0, SP)])
  pltpu.sync_copy(degi_sh.at[pl.ds(r0, SP)], zbuf)
  pltpu.sync_copy(zbuf, out_hbm.at[pl.ds(obase + NP + r0, SP)])


@functools.partial(
    pl.kernel,
    out_type=jax.ShapeDtypeStruct((NC, NP, D), jnp.float32),
    mesh=_sc_mesh(),
    scratch_types=[
        pltpu.VMEM((2, ECH), jnp.int32),
        pltpu.VMEM((2, ECH), jnp.int32),
        pltpu.VMEM((2, ECH), jnp.int32),
        pltpu.VMEM((2, ECH), jnp.int32),
        pltpu.VMEM((ECH, D), jnp.float32),
        pltpu.VMEM((ECH, D), jnp.float32),
        pltpu.VMEM_SHARED((NP, D), jnp.float32),
        pltpu.SemaphoreType.DMA,
        pltpu.SemaphoreType.DMA,
        pltpu.SemaphoreType.DMA,
        pltpu.SemaphoreType.DMA,
        pltpu.SemaphoreType.DMA,
        pltpu.SemaphoreType.DMA,
        pltpu.SemaphoreType.DMA,
        pltpu.SemaphoreType.DMA,
    ],
)
def _agg_sc(h_hbm, tbl_hbm, zeros_hbm, out_hbm,
            i_a, i_b, d_a, d_b, r_a, r_b, agg_sh,
            s_ia, s_ib, s_ga, s_gb, s_sa, s_sb, s_da, s_db):
  cid = lax.axis_index("c")
  sid = lax.axis_index("s")
  wid = cid * NS + sid
  ench = jnp.where(cid == 0, ENCH0, ENCH1)
  r0 = sid * SP
  pltpu.sync_copy(zeros_hbm.at[pl.ds(r0, SP)], agg_sh.at[pl.ds(r0, SP)])
  plsc.subcore_barrier()

  pltpu.async_copy(tbl_hbm.at[wid, 0], i_a, s_ia).wait()
  pltpu.async_copy(h_hbm.at[i_a.at[0]], r_a, s_ga)
  pltpu.async_copy(tbl_hbm.at[wid, 0], d_a, s_da)
  pltpu.async_copy(tbl_hbm.at[wid, 1], i_b, s_ib)

  def chunk(j, ic, dc, rc, sic, sgc, ssc, sdc, io, do, ro, sio, sgo, sso,
            sdo):
    pltpu.make_async_copy(h_hbm.at[ic.at[0]], rc, sgc).wait()

    @pl.when(j + 2 < ench)
    def _():
      pltpu.async_copy(tbl_hbm.at[wid, j + 2], ic, sic)

    @pl.when(j + 1 < ench)
    def _():
      @pl.when(j >= 1)
      def _():
        pltpu.make_async_copy(ro, agg_sh.at[do.at[1]], sso).wait()

      pltpu.make_async_copy(tbl_hbm.at[wid, j + 1], io, sio).wait()
      pltpu.async_copy(h_hbm.at[io.at[0]], ro, sgo)
      pltpu.async_copy(tbl_hbm.at[wid, j + 1], do, sdo)

    pltpu.make_async_copy(tbl_hbm.at[wid, j], dc, sdc).wait()
    pltpu.async_copy(rc, agg_sh.at[dc.at[1]], ssc, add=True)

  def pair(i, carry):
    j = 2 * i
    chunk(j, i_a, d_a, r_a, s_ia, s_ga, s_sa, s_da,
          i_b, d_b, r_b, s_ib, s_gb, s_sb, s_db)
    chunk(j + 1, i_b, d_b, r_b, s_ib, s_gb, s_sb, s_db,
          i_a, d_a, r_a, s_ia, s_ga, s_sa, s_da)
    return carry

  lax.fori_loop(0, ench // 2, pair, 0)

  @pl.when(ench % 2 == 1)
  def _():
    chunk(ench - 1, i_a, d_a, r_a, s_ia, s_ga, s_sa, s_da,
          i_b, d_b, r_b, s_ib, s_gb, s_sb, s_db)

  pltpu.make_async_copy(r_a, agg_sh.at[d_a.at[1]], s_sa).wait()
  pltpu.make_async_copy(r_b, agg_sh.at[d_b.at[1]], s_sb).wait()
  plsc.subcore_barrier()
  pltpu.sync_copy(agg_sh.at[pl.ds(r0, SP)], out_hbm.at[cid, pl.ds(r0, SP)])


@functools.partial(
    pl.kernel,
    out_type=(jax.ShapeDtypeStruct((B, D), jnp.float32),
              jax.ShapeDtypeStruct((B, D), jnp.float32)),
    mesh=_sc_mesh(),
    scratch_types=[
        pltpu.VMEM((2 * BNCH, BCH), jnp.int32),
        pltpu.VMEM((BCH, D), jnp.float32),
        pltpu.VMEM((BCH, D), jnp.float32),
        pltpu.VMEM((BCH, D), jnp.float32),
        pltpu.VMEM((BCH, D), jnp.float32),
        pltpu.SemaphoreType.DMA,
        pltpu.SemaphoreType.DMA,
        pltpu.SemaphoreType.DMA,
        pltpu.SemaphoreType.DMA,
    ],
)
def _pair_sc(g_hbm, tidx_hbm, o_out, d_out,
             ti_v, ro_a, ro_b, rd_a, rd_b, soa, sob, sda, sdb):
  cid = lax.axis_index("c")
  sid = lax.axis_index("s")
  wid = sid * NC + cid
  base = wid * BPW
  pltpu.sync_copy(tidx_hbm.at[wid], ti_v)
  ro = (ro_a, ro_b)
  rd = (rd_a, rd_b)
  so = (soa, sob)
  sd = (sda, sdb)
  pltpu.async_copy(g_hbm.at[ti_v.at[0]], ro_a, soa)
  pltpu.async_copy(g_hbm.at[ti_v.at[1]], rd_a, sda)
  for j in range(BNCH):
    k = j % 2
    kn = (j + 1) % 2
    pltpu.make_async_copy(g_hbm.at[ti_v.at[2 * j]], ro[k], so[k]).wait()
    if j + 1 < BNCH:
      pltpu.async_copy(g_hbm.at[ti_v.at[2 * j + 2]], ro[kn], so[kn])
    pltpu.sync_copy(ro[k], o_out.at[pl.ds(base + j * BCH, BCH)])
    pltpu.make_async_copy(g_hbm.at[ti_v.at[2 * j + 1]], rd[k], sd[k]).wait()
    if j + 1 < BNCH:
      pltpu.async_copy(g_hbm.at[ti_v.at[2 * j + 3]], rd[kn], sd[kn])
    pltpu.sync_copy(rd[k], d_out.at[pl.ds(base + j * BCH, BCH)])


def _scale_body(x_ref, s_ref, o_ref):
  o_ref[...] = x_ref[...] * s_ref[...]


def _scale_tc(x, s):
  return pl.pallas_call(
      _scale_body,
      grid=(N // BN,),
      in_specs=[pl.BlockSpec((BN, D), lambda i: (i, 0)),
                pl.BlockSpec((BN, 1), lambda i: (i, 0))],
      out_specs=pl.BlockSpec((BN, D), lambda i: (i, 0)),
      out_shape=jax.ShapeDtypeStruct((NP, D), jnp.float32),
  )(x, s)


def _layer_mid_body(ap, sin, w, b, sout, o_ref):
  t = (ap[0] + ap[1]) * sin[...]
  z = jnp.dot(t, w[...], preferred_element_type=jnp.float32) + b[...]
  o_ref[...] = jnp.maximum(z, 0.0) * sout[...]


def _layer_last_body(ap, sin, w, b, o_ref):
  t = (ap[0] + ap[1]) * sin[...]
  z = jnp.dot(t, w[...], preferred_element_type=jnp.float32) + b[...]
  o_ref[...] = jax.nn.sigmoid(z)


def _layer_tc(ap, sin, w, b, sout):
  blk = pl.BlockSpec((BN, D), lambda i: (i, 0))
  col = pl.BlockSpec((BN, 1), lambda i: (i, 0))
  specs = [pl.BlockSpec((NC, BN, D), lambda i: (0, i, 0)), col,
           pl.BlockSpec((D, D), lambda i: (0, 0)),
           pl.BlockSpec((1, D), lambda i: (0, 0))]
  if sout is None:
    return pl.pallas_call(
        _layer_last_body,
        grid=(N // BN,),
        in_specs=specs,
        out_specs=blk,
        out_shape=jax.ShapeDtypeStruct((NP, D), jnp.float32),
    )(ap, sin, w, b)
  return pl.pallas_call(
      _layer_mid_body,
      grid=(N // BN,),
      in_specs=specs + [col],
      out_specs=blk,
      out_shape=jax.ShapeDtypeStruct((NP, D), jnp.float32),
  )(ap, sin, w, b, sout)


def _final_body(o_in, d_in, dv, w_ref, b_ref, out_ref):
  w = w_ref[...]
  z = jnp.dot(o_in[...], w[0:D, :], preferred_element_type=jnp.float32)
  z = z + jnp.dot(d_in[...], w[D:2 * D, :], preferred_element_type=jnp.float32)
  z = z + dv[...] * w[2 * D:2 * D + 1, :] + b_ref[...]
  out_ref[...] = jnp.tanh(z)


def _final_tc(o_emb, d_emb, disv, w_lin, b_lin):
  bb = 4096
  return pl.pallas_call(
      _final_body,
      grid=(B // bb,),
      in_specs=[
          pl.BlockSpec((bb, D), lambda i: (i, 0)),
          pl.BlockSpec((bb, D), lambda i: (i, 0)),
          pl.BlockSpec((bb, 1), lambda i: (i, 0)),
          pl.BlockSpec((2 * D + 1, 1), lambda i: (0, 0)),
          pl.BlockSpec((1, 1), lambda i: (0, 0)),
      ],
      out_specs=pl.BlockSpec((bb, 1), lambda i: (i, 0)),
      out_shape=jax.ShapeDtypeStruct((B, 1), jnp.float32),
  )(o_emb, d_emb, disv, w_lin, b_lin)


def kernel(x, edge_index, dis, train_idx, W_in, b_in, W_hid, b_hid,
           W_out, b_out, W_lin, b_lin):
  def mk_side(v):
    a = v[:E0].reshape(NS, ENCH0, ECH)
    a = jnp.pad(a, ((0, 0), (0, ENCH - ENCH0), (0, 0)),
                constant_values=NP - 1)
    b = jnp.pad(v[E0:], (0, NS * ENCH1 * ECH - (E - E0)),
                constant_values=NP - 1).reshape(NS, ENCH1, ECH)
    b = jnp.pad(b, ((0, 0), (0, ENCH - ENCH1), (0, 0)),
                constant_values=NP - 1)
    return jnp.concatenate([a, b], axis=0)

  tbl = jnp.stack([mk_side(edge_index[0]), mk_side(edge_index[1])],
                  axis=2)
  tidx = jnp.stack([train_idx[0].reshape(NW, BNCH, BCH),
                    train_idx[1].reshape(NW, BNCH, BCH)],
                   axis=2).reshape(NW, 2 * BNCH, BCH)

  zeros_nd = jnp.zeros((NP, D), jnp.float32)
  zeros_n = jnp.zeros((NP,), jnp.float32)
  ones_e = jnp.ones((ECH,), jnp.float32)

  tbl2 = tbl.reshape(NW, 2 * ENCH, ECH)
  degp = _deg_sc(tbl2, ones_e, zeros_n).reshape(NC, 2, NP)
  deg = degp[0] + degp[1]
  inv = lax.rsqrt(jnp.maximum(deg, 1.0))
  inv_out_c = inv[0][:, None]
  inv_in_c = inv[1][:, None]

  h = _scale_tc(x, inv_out_c)
  for w, b, last in ((W_in, b_in, False), (W_hid, b_hid, False),
                     (W_out, b_out, True)):
    aggp = _agg_sc(h, tbl, zeros_nd)
    h = _layer_tc(aggp, inv_in_c, w, b.reshape(1, D),
                  None if last else inv_out_c)

  o_emb, d_emb = _pair_sc(h, tidx)
  disv = dis[train_idx[0], train_idx[1]]
  return _final_tc(o_emb, d_emb, disv.reshape(B, 1), W_lin,
                   b_lin.reshape(1, 1))


# speedup vs baseline: 1.0527x; 1.0527x over previous
import functools

import jax
import jax.numpy as jnp
from jax import lax
from jax.experimental import pallas as pl
from jax.experimental.pallas import tpu as pltpu
from jax.experimental.pallas import tpu_sc as plsc

N = 10000
E = 320000
D = 128
B = 16384

NC = 2
NS = 16
NW = NC * NS
SP = 632
NP = NS * SP

ECH = 128
ENCH0 = 100
ENCH1 = 57
ENCH = max(ENCH0, ENCH1)
E0 = NS * ENCH0 * ECH
EPAD = NW * ENCH * ECH

BPW = B // NW
BCH = 128
BNCH = BPW // BCH

BN = 2528


def _sc_mesh():
  return plsc.VectorSubcoreMesh(
      core_axis_name="c", subcore_axis_name="s",
      num_cores=NC, num_subcores=NS)


@functools.partial(
    pl.kernel,
    out_type=jax.ShapeDtypeStruct((NC * 2 * NP,), jnp.float32),
    mesh=_sc_mesh(),
    scratch_types=[
        pltpu.VMEM((2 * ENCH, ECH), jnp.int32),
        pltpu.VMEM((ECH,), jnp.float32),
        pltpu.VMEM((SP,), jnp.float32),
        pltpu.VMEM_SHARED((NP,), jnp.float32),
        pltpu.VMEM_SHARED((NP,), jnp.float32),
        pltpu.SemaphoreType.DMA,
        pltpu.SemaphoreType.DMA,
    ],
)
def _deg_sc(tbl_hbm, ones_hbm, zeros_hbm, out_hbm,
            tb_v, ones_v, zbuf, dego_sh, degi_sh, s_o, s_i):
  cid = lax.axis_index("c")
  sid = lax.axis_index("s")
  wid = cid * NS + sid
  ench = jnp.where(cid == 0, ENCH0, ENCH1)
  r0 = sid * SP
  pltpu.sync_copy(zeros_hbm.at[pl.ds(0, SP)], zbuf)
  pltpu.sync_copy(zbuf, dego_sh.at[pl.ds(r0, SP)])
  pltpu.sync_copy(zbuf, degi_sh.at[pl.ds(r0, SP)])
  pltpu.sync_copy(tbl_hbm.at[wid], tb_v)
  pltpu.sync_copy(ones_hbm, ones_v)
  plsc.subcore_barrier()

  def body(j, carry):
    a = pltpu.async_copy(ones_v, dego_sh.at[tb_v.at[2 * j]], s_o, add=True)
    b = pltpu.async_copy(ones_v, degi_sh.at[tb_v.at[2 * j + 1]], s_i,
                         add=True)
    a.wait()
    b.wait()
    return carry

  lax.fori_loop(0, ench, body, 0)
  plsc.subcore_barrier()
  obase = cid * 2 * NP
  pltpu.sync_copy(dego_sh.at[pl.ds(r0, SP)], zbuf)
  pltpu.sync_copy(zbuf, out_hbm.at[pl.ds(obase + r0, SP)])
  pltpu.sync_copy(degi_sh.at[pl.ds(r0, SP)], zbuf)
  pltpu.sync_copy(zbuf, out_hbm.at[pl.ds(obase + NP + r0, SP)])


@functools.partial(
    pl.kernel,
    out_type=jax.ShapeDtypeStruct((NC, NP, D), jnp.float32),
    mesh=_sc_mesh(),
    scratch_types=[
        pltpu.VMEM((2, ECH), jnp.int32),
        pltpu.VMEM((2, ECH), jnp.int32),
        pltpu.VMEM((2, ECH), jnp.int32),
        pltpu.VMEM((2, ECH), jnp.int32),
        pltpu.VMEM((ECH, D), jnp.float32),
        pltpu.VMEM((ECH, D), jnp.float32),
        pltpu.VMEM_SHARED((NP, D), jnp.float32),
        pltpu.SemaphoreType.DMA,
        pltpu.SemaphoreType.DMA,
        pltpu.SemaphoreType.DMA,
        pltpu.SemaphoreType.DMA,
        pltpu.SemaphoreType.DMA,
        pltpu.SemaphoreType.DMA,
        pltpu.SemaphoreType.DMA,
        pltpu.SemaphoreType.DMA,
    ],
)
def _agg_sc(h_hbm, tbl_hbm, zeros_hbm, out_hbm,
            i_a, i_b, d_a, d_b, r_a, r_b, agg_sh,
            s_ia, s_ib, s_ga, s_gb, s_sa, s_sb, s_da, s_db):
  cid = lax.axis_index("c")
  sid = lax.axis_index("s")
  wid = cid * NS + sid
  ench = jnp.where(cid == 0, ENCH0, ENCH1)
  r0 = sid * SP
  pltpu.sync_copy(zeros_hbm.at[pl.ds(r0, SP)], agg_sh.at[pl.ds(r0, SP)])
  plsc.subcore_barrier()

  pltpu.async_copy(tbl_hbm.at[wid, 0], i_a, s_ia).wait()
  pltpu.async_copy(h_hbm.at[i_a.at[0]], r_a, s_ga)
  pltpu.async_copy(tbl_hbm.at[wid, 0], d_a, s_da)
  pltpu.async_copy(tbl_hbm.at[wid, 1], i_b, s_ib)

  def chunk(j, ic, dc, rc, sic, sgc, ssc, sdc, io, do, ro, sio, sgo, sso,
            sdo):
    pltpu.make_async_copy(h_hbm.at[ic.at[0]], rc, sgc).wait()

    @pl.when(j + 2 < ench)
    def _():
      pltpu.async_copy(tbl_hbm.at[wid, j + 2], ic, sic)

    @pl.when(j + 1 < ench)
    def _():
      @pl.when(j >= 1)
      def _():
        pltpu.make_async_copy(ro, agg_sh.at[do.at[1]], sso).wait()

      pltpu.make_async_copy(tbl_hbm.at[wid, j + 1], io, sio).wait()
      pltpu.async_copy(h_hbm.at[io.at[0]], ro, sgo)
      pltpu.async_copy(tbl_hbm.at[wid, j + 1], do, sdo)

    pltpu.make_async_copy(tbl_hbm.at[wid, j], dc, sdc).wait()
    pltpu.async_copy(rc, agg_sh.at[dc.at[1]], ssc, add=True)

  def pair(i, carry):
    j = 2 * i
    chunk(j, i_a, d_a, r_a, s_ia, s_ga, s_sa, s_da,
          i_b, d_b, r_b, s_ib, s_gb, s_sb, s_db)
    chunk(j + 1, i_b, d_b, r_b, s_ib, s_gb, s_sb, s_db,
          i_a, d_a, r_a, s_ia, s_ga, s_sa, s_da)
    return carry

  lax.fori_loop(0, ench // 2, pair, 0)

  @pl.when(ench % 2 == 1)
  def _():
    chunk(ench - 1, i_a, d_a, r_a, s_ia, s_ga, s_sa, s_da,
          i_b, d_b, r_b, s_ib, s_gb, s_sb, s_db)

  pltpu.make_async_copy(r_a, agg_sh.at[d_a.at[1]], s_sa).wait()
  pltpu.make_async_copy(r_b, agg_sh.at[d_b.at[1]], s_sb).wait()
  plsc.subcore_barrier()
  pltpu.sync_copy(agg_sh.at[pl.ds(r0, SP)], out_hbm.at[cid, pl.ds(r0, SP)])


@functools.partial(
    pl.kernel,
    out_type=(jax.ShapeDtypeStruct((B, D), jnp.float32),
              jax.ShapeDtypeStruct((B, D), jnp.float32)),
    mesh=_sc_mesh(),
    scratch_types=[
        pltpu.VMEM((2, BCH), jnp.int32),
        pltpu.VMEM((BCH, D), jnp.float32),
        pltpu.VMEM((BCH, D), jnp.float32),
        pltpu.SemaphoreType.DMA,
        pltpu.SemaphoreType.DMA,
    ],
)
def _pair_sc(g_hbm, tidx_hbm,
             o_out, d_out, idx_v, rows_o, rows_d, sem_o, sem_d):
  cid = lax.axis_index("c")
  sid = lax.axis_index("s")
  wid = sid * NC + cid
  base = wid * BPW
  for j in range(BNCH):
    pltpu.sync_copy(tidx_hbm.at[wid, j], idx_v)
    a = pltpu.async_copy(g_hbm.at[idx_v.at[0]], rows_o, sem_o)
    b = pltpu.async_copy(g_hbm.at[idx_v.at[1]], rows_d, sem_d)
    a.wait()
    pltpu.sync_copy(rows_o, o_out.at[pl.ds(base + j * BCH, BCH)])
    b.wait()
    pltpu.sync_copy(rows_d, d_out.at[pl.ds(base + j * BCH, BCH)])


def _scale_body(x_ref, s_ref, o_ref):
  o_ref[...] = x_ref[...] * s_ref[...]


def _scale_tc(x, s):
  return pl.pallas_call(
      _scale_body,
      grid=(NP // BN,),
      in_specs=[pl.BlockSpec((BN, D), lambda i: (i, 0))] * 2,
      out_specs=pl.BlockSpec((BN, D), lambda i: (i, 0)),
      out_shape=jax.ShapeDtypeStruct((NP, D), jnp.float32),
  )(x, s)


def _layer_mid_body(ap, sin, w, b, sout, o_ref):
  t = (ap[0] + ap[1]) * sin[...]
  z = jnp.dot(t, w[...], preferred_element_type=jnp.float32) + b[...]
  o_ref[...] = jnp.maximum(z, 0.0) * sout[...]


def _layer_last_body(ap, sin, w, b, o_ref):
  t = (ap[0] + ap[1]) * sin[...]
  z = jnp.dot(t, w[...], preferred_element_type=jnp.float32) + b[...]
  o_ref[...] = jax.nn.sigmoid(z)


def _layer_tc(ap, sin, w, b, sout):
  blk = pl.BlockSpec((BN, D), lambda i: (i, 0))
  specs = [pl.BlockSpec((NC, BN, D), lambda i: (0, i, 0)), blk,
           pl.BlockSpec((D, D), lambda i: (0, 0)),
           pl.BlockSpec((1, D), lambda i: (0, 0))]
  if sout is None:
    return pl.pallas_call(
        _layer_last_body,
        grid=(NP // BN,),
        in_specs=specs,
        out_specs=blk,
        out_shape=jax.ShapeDtypeStruct((NP, D), jnp.float32),
    )(ap, sin, w, b)
  return pl.pallas_call(
      _layer_mid_body,
      grid=(NP // BN,),
      in_specs=specs + [blk],
      out_specs=blk,
      out_shape=jax.ShapeDtypeStruct((NP, D), jnp.float32),
  )(ap, sin, w, b, sout)


def _final_body(o_in, d_in, dv, w_ref, b_ref, out_ref):
  w = w_ref[...]
  z = jnp.dot(o_in[...], w[0:D, :], preferred_element_type=jnp.float32)
  z = z + jnp.dot(d_in[...], w[D:2 * D, :], preferred_element_type=jnp.float32)
  z = z + dv[...] * w[2 * D:2 * D + 1, :] + b_ref[...]
  out_ref[...] = jnp.tanh(z)


def _final_tc(o_emb, d_emb, disv, w_lin, b_lin):
  bb = 2048
  return pl.pallas_call(
      _final_body,
      grid=(B // bb,),
      in_specs=[
          pl.BlockSpec((bb, D), lambda i: (i, 0)),
          pl.BlockSpec((bb, D), lambda i: (i, 0)),
          pl.BlockSpec((bb, 1), lambda i: (i, 0)),
          pl.BlockSpec((2 * D + 1, 1), lambda i: (0, 0)),
          pl.BlockSpec((1, 1), lambda i: (0, 0)),
      ],
      out_specs=pl.BlockSpec((bb, 1), lambda i: (i, 0)),
      out_shape=jax.ShapeDtypeStruct((B, 1), jnp.float32),
  )(o_emb, d_emb, disv, w_lin, b_lin)


def kernel(x, edge_index, dis, train_idx, W_in, b_in, W_hid, b_hid,
           W_out, b_out, W_lin, b_lin):
  def mk_side(v):
    a = v[:E0].reshape(NS, ENCH0, ECH)
    a = jnp.pad(a, ((0, 0), (0, ENCH - ENCH0), (0, 0)),
                constant_values=NP - 1)
    b = jnp.pad(v[E0:], (0, NS * ENCH1 * ECH - (E - E0)),
                constant_values=NP - 1).reshape(NS, ENCH1, ECH)
    b = jnp.pad(b, ((0, 0), (0, ENCH - ENCH1), (0, 0)),
                constant_values=NP - 1)
    return jnp.concatenate([a, b], axis=0)

  tbl = jnp.stack([mk_side(edge_index[0]), mk_side(edge_index[1])],
                  axis=2)
  tidx = jnp.stack([train_idx[0].reshape(NW, BNCH, BCH),
                    train_idx[1].reshape(NW, BNCH, BCH)],
                   axis=2)

  xp = jnp.pad(x, ((0, NP - N), (0, 0)))
  zeros_nd = jnp.zeros((NP, D), jnp.float32)
  zeros_n = jnp.zeros((NP,), jnp.float32)
  ones_e = jnp.ones((ECH,), jnp.float32)

  tbl2 = tbl.reshape(NW, 2 * ENCH, ECH)
  degp = _deg_sc(tbl2, ones_e, zeros_n).reshape(NC, 2, NP)
  deg = degp[0] + degp[1]
  inv = lax.rsqrt(jnp.maximum(deg, 1.0))
  inv_out_b = jnp.broadcast_to(inv[0][:, None], (NP, D))
  inv_in_b = jnp.broadcast_to(inv[1][:, None], (NP, D))

  h = _scale_tc(xp, inv_out_b)
  for w, b, last in ((W_in, b_in, False), (W_hid, b_hid, False),
                     (W_out, b_out, True)):
    aggp = _agg_sc(h, tbl, zeros_nd)
    h = _layer_tc(aggp, inv_in_b, w, b.reshape(1, D),
                  None if last else inv_out_b)

  o_emb, d_emb = _pair_sc(h, tidx)
  disv = dis[train_idx[0], train_idx[1]]
  return _final_tc(o_emb, d_emb, disv.reshape(B, 1), W_lin,
                   b_lin.reshape(1, 1))
